# baseline (speedup 1.0000x reference)
import jax
import jax.numpy as jnp
from jax.experimental import pallas as pl
from jax.experimental.pallas import tpu as pltpu

ORIGINAL_DIM = 4096
PROJ_DIM = 1024
C = 4
BATCH = 2048

BM = 512


_DOT_DIMS = (((1,), (0,)), ((), ()))


def _sjlt_kernel(idx_ref, sign_ref, x_ref, o_ref, s_ref):
    @pl.when(pl.program_id(0) == 0)
    def _build_and_project():
        DB = 128
        p = jax.lax.broadcasted_iota(jnp.int16, (DB, PROJ_DIM), 1)
        acc_out = jnp.zeros((BM, PROJ_DIM), jnp.float32)
        for d0 in range(0, ORIGINAL_DIM, DB):
            idx = idx_ref[d0:d0 + DB, :]
            sign = sign_ref[d0:d0 + DB, :]
            s_chunk = jnp.where(idx[:, 0][:, None] == p,
                                sign[:, 0][:, None], jnp.bfloat16(0.0))
            for j in range(1, C):
                s_chunk += jnp.where(idx[:, j][:, None] == p,
                                     sign[:, j][:, None],
                                     jnp.bfloat16(0.0))
            s_ref[d0:d0 + DB, :] = s_chunk
            acc_out += jax.lax.dot_general(
                x_ref[:, d0:d0 + DB], s_chunk, _DOT_DIMS,
                preferred_element_type=jnp.float32)
        o_ref[...] = acc_out

    @pl.when(pl.program_id(0) > 0)
    def _project():
        o_ref[...] = jax.lax.dot_general(
            x_ref[...], s_ref[...], _DOT_DIMS,
            preferred_element_type=jnp.float32)


@jax.jit
def kernel(x, rand_indices, rand_signs):
    idx = rand_indices.astype(jnp.int16)
    sign = rand_signs.astype(jnp.bfloat16)
    grid = (BATCH // BM,)
    return pl.pallas_call(
        _sjlt_kernel,
        grid=grid,
        in_specs=[
            pl.BlockSpec((ORIGINAL_DIM, C), lambda i: (0, 0)),
            pl.BlockSpec((ORIGINAL_DIM, C), lambda i: (0, 0)),
            pl.BlockSpec((BM, ORIGINAL_DIM), lambda i: (i, 0)),
        ],
        out_specs=pl.BlockSpec((BM, PROJ_DIM), lambda i: (i, 0)),
        out_shape=jax.ShapeDtypeStruct((BATCH, PROJ_DIM), jnp.float32),
        scratch_shapes=[pltpu.VMEM((ORIGINAL_DIM, PROJ_DIM), jnp.bfloat16)],
    )(idx, sign, x)


# speedup vs baseline: 1.1027x; 1.1027x over previous
import jax
import jax.numpy as jnp
from jax.experimental import pallas as pl
from jax.experimental.pallas import tpu as pltpu

ORIGINAL_DIM = 4096
PROJ_DIM = 1024
C = 4
BATCH = 2048

BM = 512


_DOT_DIMS = (((1,), (0,)), ((), ()))


def _sjlt_kernel(idx_ref, sign_ref, x_ref, o_ref, s_ref):
    @pl.when(pl.program_id(0) == 0)
    def _build_and_project():
        DB = 256
        p = jax.lax.broadcasted_iota(jnp.int16, (DB, PROJ_DIM), 1)
        acc_out = jnp.zeros((BM, PROJ_DIM), jnp.float32)
        for d0 in range(0, ORIGINAL_DIM, DB):
            idx = idx_ref[d0:d0 + DB, :]
            sign = sign_ref[d0:d0 + DB, :]
            s_chunk = jnp.where(idx[:, 0][:, None] == p,
                                sign[:, 0][:, None], jnp.bfloat16(0.0))
            for j in range(1, C):
                s_chunk += jnp.where(idx[:, j][:, None] == p,
                                     sign[:, j][:, None],
                                     jnp.bfloat16(0.0))
            s_ref[d0:d0 + DB, :] = s_chunk
            acc_out += jax.lax.dot_general(
                x_ref[:, d0:d0 + DB], s_chunk, _DOT_DIMS,
                preferred_element_type=jnp.float32)
        o_ref[...] = acc_out

    @pl.when(pl.program_id(0) > 0)
    def _project():
        o_ref[...] = jax.lax.dot_general(
            x_ref[...], s_ref[...], _DOT_DIMS,
            preferred_element_type=jnp.float32)


@jax.jit
def kernel(x, rand_indices, rand_signs):
    idx = rand_indices.astype(jnp.int16)
    sign = rand_signs.astype(jnp.bfloat16)
    grid = (BATCH // BM,)
    return pl.pallas_call(
        _sjlt_kernel,
        grid=grid,
        in_specs=[
            pl.BlockSpec((ORIGINAL_DIM, C), lambda i: (0, 0)),
            pl.BlockSpec((ORIGINAL_DIM, C), lambda i: (0, 0)),
            pl.BlockSpec((BM, ORIGINAL_DIM), lambda i: (i, 0)),
        ],
        out_specs=pl.BlockSpec((BM, PROJ_DIM), lambda i: (i, 0)),
        out_shape=jax.ShapeDtypeStruct((BATCH, PROJ_DIM), jnp.float32),
        scratch_shapes=[pltpu.VMEM((ORIGINAL_DIM, PROJ_DIM), jnp.bfloat16)],
    )(idx, sign, x)
